# baseline (speedup 1.0000x reference)
import functools

import numpy as np

import jax
import jax.numpy as jnp
from jax import lax
from jax.experimental import pallas as pl
from jax.experimental.pallas import tpu as pltpu
from jax.experimental.pallas import tpu_sc as plsc

B, T, C = 64, 1024, 768
KEEP = 256
LANES = 16
NC, NS = 2, 16
NW = NC * NS
BPW = B // NW
CHUNK = 32
NBUF = 4
NCH = BPW * KEEP // CHUNK


_FWD_NP = None


def _forward_indexes_np():
    global _FWD_NP
    if _FWD_NP is None:
        with jax.ensure_compile_time_eval(), \
             jax.default_device(jax.devices("cpu")[0]):
            keys = jax.random.split(jax.random.key(42), B)
            fwd = jnp.stack(
                [jax.random.permutation(k, T) for k in keys], axis=0)
            _FWD_NP = np.asarray(fwd)
    return _FWD_NP


def _gather_indexes_np():
    fwd = _forward_indexes_np()
    g = fwd[:, :KEEP] + (np.arange(B, dtype=np.int32) * T)[:, None]
    return np.ascontiguousarray(g.reshape(NW, NCH, CHUNK))


@functools.cache
def _build_shuffle_kernel():
    return pl.kernel(
        _shuffle_body,
        mesh=plsc.VectorSubcoreMesh(core_axis_name="c", subcore_axis_name="s"),
        compiler_params=pltpu.CompilerParams(needs_layout_passes=False),
        out_type=[
            jax.ShapeDtypeStruct((B * KEEP, C), jnp.float32),
            jax.ShapeDtypeStruct((B, T), jnp.int32),
        ],
        scratch_types=[
            pltpu.VMEM((BPW, T), jnp.int32),
            pltpu.VMEM((BPW * T,), jnp.int32),
            pltpu.VMEM((NCH, CHUNK), jnp.int32),
            pltpu.VMEM((NBUF, CHUNK, C), jnp.float32),
            [pltpu.SemaphoreType.DMA] * NBUF,
            [pltpu.SemaphoreType.DMA] * NBUF,
            pltpu.SemaphoreType.DMA,
            pltpu.SemaphoreType.DMA,
        ],
    )


def _shuffle_body(flat_hbm, gidx_hbm, fwd_hbm, out_hbm, bwd_hbm,
                  fwd_v, bwd_v, idx_v, buf_v, gsem, wsem, fsem, bsem):
    wid = lax.axis_index("s") * NC + lax.axis_index("c")
    base = wid * BPW * KEEP
    gcp = [None] * NCH
    wcp = [None] * NCH

    def issue_gather(ch):
        return pltpu.async_copy(
            flat_hbm.at[idx_v.at[ch]], buf_v.at[ch % NBUF], gsem[ch % NBUF])

    def issue_write(ch):
        return pltpu.async_copy(
            buf_v.at[ch % NBUF],
            out_hbm.at[pl.ds(base + ch * CHUNK, CHUNK)], wsem[ch % NBUF])

    pltpu.sync_copy(gidx_hbm.at[wid], idx_v)
    for ch in range(NBUF):
        gcp[ch] = issue_gather(ch)

    fcp = [pltpu.async_copy(fwd_hbm.at[wid * BPW + j], fwd_v.at[j], fsem)
           for j in range(BPW)]
    for cp in fcp:
        cp.wait()
    bcp = []
    for j in range(BPW):
        for k in range(T // LANES):
            plsc.store_scatter(bwd_v,
                               [fwd_v[j, pl.ds(k * LANES, LANES)] + j * T],
                               lax.iota(jnp.int32, LANES) + k * LANES)
        bcp.append(pltpu.async_copy(
            bwd_v.at[pl.ds(j * T, T)], bwd_hbm.at[wid * BPW + j], bsem))

    for ch in range(NCH):
        gcp[ch].wait()
        wcp[ch] = issue_write(ch)
        nxt = ch + NBUF
        if nxt < NCH:
            wcp[ch].wait()
            gcp[nxt] = issue_gather(nxt)
    for ch in range(NCH - NBUF, NCH):
        wcp[ch].wait()
    for cp in bcp:
        cp.wait()


def kernel(patches):
    b, t, c = patches.shape
    fwd = jnp.asarray(_forward_indexes_np())
    gidx = jnp.asarray(_gather_indexes_np())
    flat = patches.reshape(b * t, c)
    out_flat, bwd = _build_shuffle_kernel()(flat, gidx, fwd)
    return (out_flat.reshape(b, KEEP, c), fwd, bwd)


# speedup vs baseline: 1.0163x; 1.0163x over previous
import functools

import numpy as np

import jax
import jax.numpy as jnp
from jax import lax
from jax.experimental import pallas as pl
from jax.experimental.pallas import tpu as pltpu
from jax.experimental.pallas import tpu_sc as plsc

B, T, C = 64, 1024, 768
KEEP = 256
LANES = 16
NC, NS = 2, 16
NW = NC * NS
BPW = B // NW
CHUNK = 32
NBUF = 4
NCH = BPW * KEEP // CHUNK


_FWD_NP = None


def _forward_indexes_np():
    global _FWD_NP
    if _FWD_NP is None:
        with jax.ensure_compile_time_eval(), \
             jax.default_device(jax.devices("cpu")[0]):
            keys = jax.random.split(jax.random.key(42), B)
            fwd = jnp.stack(
                [jax.random.permutation(k, T) for k in keys], axis=0)
            _FWD_NP = np.asarray(fwd)
    return _FWD_NP


def _gather_indexes_np():
    fwd = _forward_indexes_np()
    g = fwd[:, :KEEP] + (np.arange(B, dtype=np.int32) * T)[:, None]
    return np.ascontiguousarray(g.reshape(NW, NCH, CHUNK))


@functools.cache
def _build_shuffle_kernel():
    return pl.kernel(
        _shuffle_body,
        mesh=plsc.VectorSubcoreMesh(core_axis_name="c", subcore_axis_name="s"),
        compiler_params=pltpu.CompilerParams(needs_layout_passes=False),
        out_type=[
            jax.ShapeDtypeStruct((B * KEEP, C), jnp.float32),
            jax.ShapeDtypeStruct((B, T), jnp.int32),
        ],
        scratch_types=[
            pltpu.VMEM((BPW, T), jnp.int32),
            pltpu.VMEM((BPW * T,), jnp.int32),
            pltpu.VMEM((NCH, CHUNK), jnp.int32),
            pltpu.VMEM((NBUF, CHUNK, C), jnp.float32),
            [pltpu.SemaphoreType.DMA] * NBUF,
            [pltpu.SemaphoreType.DMA] * NBUF,
            pltpu.SemaphoreType.DMA,
            pltpu.SemaphoreType.DMA,
        ],
    )


def _shuffle_body(flat_hbm, gidx_hbm, fwd_hbm, out_hbm, bwd_hbm,
                  fwd_v, bwd_v, idx_v, buf_v, gsem, wsem, fsem, bsem):
    wid = lax.axis_index("s") * NC + lax.axis_index("c")
    base = wid * BPW * KEEP

    def issue_gather(ch, s):
        return pltpu.async_copy(
            flat_hbm.at[idx_v.at[ch]], buf_v.at[s], gsem[s])

    def issue_write(ch, s):
        return pltpu.async_copy(
            buf_v.at[s],
            out_hbm.at[pl.ds(base + ch * CHUNK, CHUNK)], wsem[s])

    def wait_gather(s):
        pltpu.make_async_copy(
            flat_hbm.at[idx_v.at[0]], buf_v.at[s], gsem[s]).wait()

    def wait_write(s):
        pltpu.make_async_copy(
            buf_v.at[s], out_hbm.at[pl.ds(base, CHUNK)], wsem[s]).wait()

    pltpu.sync_copy(gidx_hbm.at[wid], idx_v)
    for s in range(NBUF):
        issue_gather(s, s)

    fcp = [pltpu.async_copy(fwd_hbm.at[wid * BPW + j], fwd_v.at[j], fsem)
           for j in range(BPW)]
    for cp in fcp:
        cp.wait()
    bcp = []
    for j in range(BPW):
        @pl.loop(0, T // LANES)
        def _scatter(k):
            plsc.store_scatter(bwd_v,
                               [fwd_v[j, pl.ds(k * LANES, LANES)] + j * T],
                               lax.iota(jnp.int32, LANES) + k * LANES)
        bcp.append(pltpu.async_copy(
            bwd_v.at[pl.ds(j * T, T)], bwd_hbm.at[wid * BPW + j], bsem))

    for ch in range(NCH):
        s = ch % NBUF
        wait_gather(s)
        issue_write(ch, s)
        nxt = ch + NBUF
        if nxt < NCH:
            wait_write(s)
            issue_gather(nxt, s)
    for s in range(NBUF):
        wait_write(s)
    for cp in bcp:
        cp.wait()


def kernel(patches):
    b, t, c = patches.shape
    fwd = jnp.asarray(_forward_indexes_np())
    gidx = jnp.asarray(_gather_indexes_np())
    flat = patches.reshape(b * t, c)
    out_flat, bwd = _build_shuffle_kernel()(flat, gidx, fwd)
    return (out_flat.reshape(b, KEEP, c), fwd, bwd)
